# Initial kernel scaffold; baseline (speedup 1.0000x reference)
import functools

import jax
import jax.numpy as jnp
from jax import lax
from jax.experimental import pallas as pl
from jax.experimental.pallas import tpu as pltpu
from jax.experimental.pallas import tpu_sc as plsc

VOCAB = 100000
D = 64
B = 1024
T = 200
CHUNK = 100
CPB = T // CHUNK
NC, NS = 2, 16
NW = NC * NS
CHUNKS_PER_W = (B * CPB) // NW
VECS = D // 16


def _build():
  mesh = plsc.VectorSubcoreMesh(
      core_axis_name="c", subcore_axis_name="s", num_cores=NC, num_subcores=NS
  )

  @functools.partial(
      pl.kernel,
      mesh=mesh,
      out_type=jax.ShapeDtypeStruct((B * CPB, CHUNK, D), jnp.float32),
      scratch_types=[
          pltpu.VMEM((CHUNKS_PER_W, CHUNK), jnp.int32),
          pltpu.VMEM((T, D), jnp.float32),
          pltpu.VMEM((CHUNK, D), jnp.float32),
          pltpu.SemaphoreType.DMA,
      ],
  )
  def k(tok_hbm, idx_hbm, pos_hbm, out_hbm, idx_v, pos_v, rows_v, sem):
    wid = lax.axis_index("s") * NC + lax.axis_index("c")
    pltpu.sync_copy(idx_hbm.at[wid], idx_v)
    pltpu.sync_copy(pos_hbm, pos_v)

    def chunk_body(c, _):
      pltpu.async_copy(tok_hbm.at[idx_v.at[c]], rows_v, sem).wait()
      phase = (c % CPB) * CHUNK

      def add_body(r, _):
        for kk in range(VECS):
          sl = pl.ds(kk * 16, 16)
          rows_v[r, sl] = rows_v[r, sl] + pos_v[phase + r, sl]
        return 0

      lax.fori_loop(0, CHUNK, add_body, 0, unroll=2)
      pltpu.sync_copy(rows_v, out_hbm.at[wid * CHUNKS_PER_W + c])
      return 0

    lax.fori_loop(0, CHUNKS_PER_W, chunk_body, 0)

  return k


def kernel(idx, tok_table, pos_table):
  idx_r = idx.astype(jnp.int32).reshape(NW, CHUNKS_PER_W, CHUNK)
  out = _build()(tok_table, idx_r, pos_table)
  return out.reshape(B, T, D)


# speedup vs baseline: 1.9706x; 1.9706x over previous
import functools

import jax
import jax.numpy as jnp
from jax import lax
from jax.experimental import pallas as pl
from jax.experimental.pallas import tpu as pltpu
from jax.experimental.pallas import tpu_sc as plsc

VOCAB = 100000
D = 64
B = 1024
T = 200
CHUNK = 100
CPB = T // CHUNK
NC, NS = 2, 16
NW = NC * NS
CHUNKS_PER_W = (B * CPB) // NW
VECS = D // 16


def _build():
  mesh = plsc.VectorSubcoreMesh(
      core_axis_name="c", subcore_axis_name="s", num_cores=NC, num_subcores=NS
  )

  @functools.partial(
      pl.kernel,
      mesh=mesh,
      compiler_params=pltpu.CompilerParams(use_tc_tiling_on_sc=False),
      out_type=jax.ShapeDtypeStruct((B * CPB, CHUNK, D), jnp.float32),
      scratch_types=[
          pltpu.VMEM((CHUNKS_PER_W, CHUNK), jnp.int32),
          pltpu.VMEM((T, D), jnp.float32),
          pltpu.VMEM((CHUNK, D), jnp.float32),
          pltpu.SemaphoreType.DMA,
      ],
  )
  def k(tok_hbm, idx_hbm, pos_hbm, out_hbm, idx_v, pos_v, rows_v, sem):
    wid = lax.axis_index("s") * NC + lax.axis_index("c")
    pltpu.sync_copy(idx_hbm.at[wid], idx_v)
    pltpu.sync_copy(pos_hbm, pos_v)

    def chunk_body(c, _):
      pltpu.async_copy(tok_hbm.at[idx_v.at[c]], rows_v, sem).wait()
      phase = (c % CPB) * CHUNK

      def add_body(r, _):
        for kk in range(VECS):
          sl = pl.ds(kk * 16, 16)
          rows_v[r, sl] = rows_v[r, sl] + pos_v[phase + r, sl]
        return 0

      lax.fori_loop(0, CHUNK, add_body, 0, unroll=2)
      pltpu.sync_copy(rows_v, out_hbm.at[wid * CHUNKS_PER_W + c])
      return 0

    lax.fori_loop(0, CHUNKS_PER_W, chunk_body, 0)

  return k


def kernel(idx, tok_table, pos_table):
  idx_r = idx.astype(jnp.int32).reshape(NW, CHUNKS_PER_W, CHUNK)
  out = _build()(tok_table, idx_r, pos_table)
  return out.reshape(B, T, D)


# speedup vs baseline: 3.2021x; 1.6250x over previous
import functools

import jax
import jax.numpy as jnp
from jax import lax
from jax.experimental import pallas as pl
from jax.experimental.pallas import tpu as pltpu
from jax.experimental.pallas import tpu_sc as plsc

VOCAB = 100000
D = 64
B = 1024
T = 200
CHUNK = 100
CPB = T // CHUNK
NC, NS = 2, 16
NW = NC * NS
CPW = (B * CPB) // NW
VECS = D // 16
NBUF = 4
LOOK = 2


def _build():
  mesh = plsc.VectorSubcoreMesh(
      core_axis_name="c", subcore_axis_name="s", num_cores=NC, num_subcores=NS
  )

  @functools.partial(
      pl.kernel,
      mesh=mesh,
      compiler_params=pltpu.CompilerParams(use_tc_tiling_on_sc=False),
      out_type=jax.ShapeDtypeStruct((B * CPB, CHUNK, D), jnp.float32),
      scratch_types=[
          pltpu.VMEM((CPW, CHUNK), jnp.int32),
          pltpu.VMEM((T, D), jnp.float32),
          pltpu.VMEM((NBUF, CHUNK, D), jnp.float32),
          pltpu.SemaphoreType.DMA((NBUF,)),
          pltpu.SemaphoreType.DMA((NBUF,)),
      ],
  )
  def k(tok_hbm, idx_hbm, pos_hbm, out_hbm, idx_v, pos_v, rows_v, gsem, ssem):
    wid = lax.axis_index("s") * NC + lax.axis_index("c")
    out0 = wid * CPW
    pltpu.sync_copy(idx_hbm.at[wid], idx_v)
    pltpu.sync_copy(pos_hbm, pos_v)

    def fire_gather(c, u):
      pltpu.async_copy(tok_hbm.at[idx_v.at[c]], rows_v.at[u], gsem.at[u])

    def wait_gather(c, u):
      pltpu.make_async_copy(
          tok_hbm.at[idx_v.at[c]], rows_v.at[u], gsem.at[u]
      ).wait()

    def fire_store(c, u):
      pltpu.async_copy(rows_v.at[u], out_hbm.at[out0 + c], ssem.at[u])

    def wait_store(u):
      pltpu.make_async_copy(
          rows_v.at[u], out_hbm.at[out0], ssem.at[u]
      ).wait()

    def add_pos(u, phase):
      buf = rows_v.at[u]

      @plsc.parallel_loop(0, CHUNK, unroll=4)
      def _(r):
        for kk in range(VECS):
          sl = pl.ds(kk * 16, 16)
          buf[r, sl] = buf[r, sl] + pos_v[phase + r, sl]

    def process(c, u, first_group):
      if not (first_group and u < LOOK):
        wait_store((u + LOOK) % NBUF)
      fire_gather(c + LOOK, (u + LOOK) % NBUF)
      wait_gather(c, u)
      add_pos(u, (u % CPB) * CHUNK)
      fire_store(c, u)

    for c0 in range(LOOK):
      fire_gather(c0, c0)

    for u in range(NBUF):
      process(u, u, True)

    def group_body(g, _):
      c_base = g * NBUF
      for u in range(NBUF):
        process(c_base + u, u, False)
      return 0

    lax.fori_loop(1, CPW // NBUF - 1, group_body, 0)

    c_base = CPW - NBUF
    for u in range(NBUF):
      c = c_base + u
      if u < NBUF - LOOK:
        wait_store((u + LOOK) % NBUF)
        fire_gather(c + LOOK, (u + LOOK) % NBUF)
      wait_gather(c, u)
      add_pos(u, (u % CPB) * CHUNK)
      fire_store(c, u)

    for u in range(NBUF):
      wait_store(u)

  return k


def kernel(idx, tok_table, pos_table):
  idx_r = idx.astype(jnp.int32).reshape(NW, CPW, CHUNK)
  out = _build()(tok_table, idx_r, pos_table)
  return out.reshape(B, T, D)


# speedup vs baseline: 6.6325x; 2.0713x over previous
import functools

import jax
import jax.numpy as jnp
from jax import lax
from jax.experimental import pallas as pl
from jax.experimental.pallas import tpu as pltpu
from jax.experimental.pallas import tpu_sc as plsc

VOCAB = 100000
D = 64
B = 1024
T = 200
BB = 128
NBT = B // BB
NCH = T * NBT
NC, NS = 2, 16
NW = NC * NS
CPW = NCH // NW
NBUF = 4
NPOS = 8
LOOK = 3


def _build():
  mesh = plsc.VectorSubcoreMesh(
      core_axis_name="c", subcore_axis_name="s", num_cores=NC, num_subcores=NS
  )

  @functools.partial(
      pl.kernel,
      mesh=mesh,
      compiler_params=pltpu.CompilerParams(
          use_tc_tiling_on_sc=False, needs_layout_passes=False
      ),
      out_type=jax.ShapeDtypeStruct((T, 8, NBT, 8, BB), jnp.float32),
      scratch_types=[
          pltpu.VMEM((CPW, BB), jnp.int32),
          pltpu.VMEM((NPOS, D), jnp.float32),
          pltpu.VMEM((NBUF, BB, D), jnp.float32),
          pltpu.VMEM((NBUF, 8, 8, BB + 1), jnp.float32),
          pltpu.SemaphoreType.DMA((NBUF,)),
          pltpu.SemaphoreType.DMA((NBUF,)),
      ],
  )
  def k(tok_hbm, idxF_hbm, pos_hbm, out_hbm,
        idx_v, pos_v, rows_v, outb, gsem, osem):
    wid = lax.axis_index("s") * NC + lax.axis_index("c")
    q0 = wid * CPW
    t0 = jnp.minimum(lax.shift_right_logical(q0, 3), T - NPOS)
    pltpu.sync_copy(idxF_hbm.at[pl.ds(q0, CPW)], idx_v)
    pltpu.sync_copy(pos_hbm.at[pl.ds(t0, NPOS)], pos_v)

    def fire_gather(j, u):
      pltpu.async_copy(tok_hbm.at[idx_v.at[j]], rows_v.at[u], gsem.at[u])

    def wait_gather(u):
      pltpu.make_async_copy(
          tok_hbm.at[idx_v.at[0]], rows_v.at[u], gsem.at[u]
      ).wait()

    def fire_store(j, u):
      q = q0 + j
      t = lax.shift_right_logical(q, 3)
      bt = lax.bitwise_and(q, NBT - 1)
      pltpu.async_copy(
          outb.at[u, :, :, pl.ds(0, BB)], out_hbm.at[t, :, bt], osem.at[u]
      )

    def wait_store(u):
      pltpu.make_async_copy(
          outb.at[u, :, :, pl.ds(0, BB)], out_hbm.at[0, :, 0], osem.at[u]
      ).wait()

    def compute(j, u):
      src = rows_v.at[u]
      dst = outb.at[u]
      tloc = lax.shift_right_logical(q0 + j, 3) - t0
      iota = lax.iota(jnp.int32, 16)
      ctv = [lax.shift_right_logical(iota + kk * 16, 3) for kk in range(4)]
      rv = [lax.bitwise_and(iota + kk * 16, 7) for kk in range(4)]
      pos = tuple(pos_v[tloc, pl.ds(kk * 16, 16)] for kk in range(4))

      @plsc.parallel_loop(0, BB, unroll=2, carry=pos)
      def _(m, pos_c):
        mv = jnp.full((16,), m, jnp.int32)
        for kk in range(4):
          v = src[m, pl.ds(kk * 16, 16)] + pos_c[kk]
          plsc.store_scatter(dst, [ctv[kk], rv[kk], mv], v)
        return pos_c

    def chunk(j, u, fire, waits):
      if fire:
        fire_gather(j + LOOK, (u + LOOK) % NBUF)
      wait_gather(u)
      if waits:
        wait_store(u)
      compute(j, u)
      fire_store(j, u)

    for j in range(LOOK):
      fire_gather(j, j)

    for j in range(4):
      chunk(j, j % NBUF, True, False)

    def group_body(g, _):
      jb = 4 + g * NBUF
      for u in range(NBUF):
        chunk(jb + u, u, True, True)
      return 0

    lax.fori_loop(0, (CPW - 4 - 6) // NBUF, group_body, 0)

    for j in range(CPW - 6, CPW):
      chunk(j, j % NBUF, j + LOOK < CPW, True)

    for u in range(NBUF):
      wait_store(u)

  return k


def kernel(idx, tok_table, pos_table):
  idxF = idx.astype(jnp.int32).T.reshape(NCH, BB)
  p5 = _build()(tok_table, idxF, pos_table)
  return p5.transpose(2, 4, 0, 1, 3).reshape(B, T, D)
